# baseline (speedup 1.0000x reference)
import functools

import jax
import jax.numpy as jnp
from jax import lax
from jax.experimental import pallas as pl
from jax.experimental.pallas import tpu as pltpu
from jax.experimental.pallas import tpu_sc as plsc

EMBD = 64
HALF = 32
CHUNK = 128
NBUF = 6


@functools.lru_cache(maxsize=None)
def _make_gather(n_half: int):
    info = plsc.get_sparse_core_info()
    nw = info.num_cores * info.num_subcores
    assert n_half % (nw * CHUNK) == 0
    chunks_per_w = n_half // (nw * CHUNK)
    rows_per_w = chunks_per_w * CHUNK

    mesh = plsc.VectorSubcoreMesh(core_axis_name="c", subcore_axis_name="s")

    @functools.partial(
        pl.kernel,
        out_type=jax.ShapeDtypeStruct((n_half, HALF), jnp.float32),
        mesh=mesh,
        scratch_types=[
            pltpu.VMEM((chunks_per_w, CHUNK), jnp.int32),
            pltpu.VMEM((NBUF, CHUNK, HALF), jnp.float32),
            pltpu.SemaphoreType.DMA((NBUF,)),
            pltpu.SemaphoreType.DMA((NBUF,)),
        ],
        compiler_params=pltpu.CompilerParams(use_tc_tiling_on_sc=False),
    )
    def gather(idx_hbm, table_hbm, out_hbm, idx_v, rows_v, gsem, osem):
        wid = lax.axis_index("s") * info.num_cores + lax.axis_index("c")
        pltpu.sync_copy(idx_hbm.at[wid], idx_v)
        out_base = wid * rows_per_w

        def start_gather(k):
            b = lax.rem(k, NBUF)
            pltpu.async_copy(table_hbm.at[idx_v.at[k]], rows_v.at[b], gsem.at[b])

        def wait_gather(k):
            b = lax.rem(k, NBUF)
            pltpu.make_async_copy(
                table_hbm.at[idx_v.at[k]], rows_v.at[b], gsem.at[b]
            ).wait()

        def out_ref(k):
            return out_hbm.at[pl.ds(out_base + k * CHUNK, CHUNK)]

        def start_out(k):
            b = lax.rem(k, NBUF)
            pltpu.async_copy(rows_v.at[b], out_ref(k), osem.at[b])

        def wait_out(k):
            b = lax.rem(k, NBUF)
            pltpu.make_async_copy(rows_v.at[b], out_ref(k), osem.at[b]).wait()

        for k in range(NBUF - 1):
            start_gather(k)

        def body(j, _):
            jn = j + NBUF - 1

            @pl.when(jnp.logical_and(jn < chunks_per_w, j > 0))
            def _():
                wait_out(j - 1)

            @pl.when(jn < chunks_per_w)
            def _():
                start_gather(jn)

            wait_gather(j)
            start_out(j)
            return 0

        lax.fori_loop(0, chunks_per_w, body, 0)

        for t in range(NBUF):
            wait_out(chunks_per_w - NBUF + t)

    return gather


def kernel(x, table):
    batch, seq = x.shape
    n_rows = batch * seq
    vocab = table.shape[0]
    info = plsc.get_sparse_core_info()
    nw = info.num_cores * info.num_subcores
    x_t = jnp.swapaxes(x, 0, 1)
    idx = x_t.astype(jnp.int32).reshape(nw, -1)
    hidx = (idx[..., None] * 2 + jnp.arange(2, dtype=jnp.int32)).reshape(
        nw, (2 * n_rows) // (nw * CHUNK), CHUNK
    )
    tbl2 = table.reshape(2 * vocab, HALF)
    out32 = _make_gather(2 * n_rows)(hidx, tbl2)
    return jnp.swapaxes(out32.reshape(seq, batch, EMBD), 0, 1)


# speedup vs baseline: 1.0062x; 1.0062x over previous
import functools

import jax
import jax.numpy as jnp
from jax import lax
from jax.experimental import pallas as pl
from jax.experimental.pallas import tpu as pltpu
from jax.experimental.pallas import tpu_sc as plsc

EMBD = 64
CHUNK = 128
NBUF = 6


@functools.lru_cache(maxsize=None)
def _make_gather(n_rows: int):
    info = plsc.get_sparse_core_info()
    nw = info.num_cores * info.num_subcores
    assert n_rows % (nw * CHUNK) == 0
    chunks_per_w = n_rows // (nw * CHUNK)
    rows_per_w = chunks_per_w * CHUNK

    mesh = plsc.VectorSubcoreMesh(core_axis_name="c", subcore_axis_name="s")

    @functools.partial(
        pl.kernel,
        out_type=jax.ShapeDtypeStruct((n_rows, EMBD), jnp.float32),
        mesh=mesh,
        scratch_types=[
            pltpu.VMEM((chunks_per_w, CHUNK), jnp.int32),
            pltpu.VMEM((NBUF, CHUNK, EMBD), jnp.float32),
            pltpu.SemaphoreType.DMA((NBUF,)),
            pltpu.SemaphoreType.DMA((NBUF,)),
        ],
        compiler_params=pltpu.CompilerParams(use_tc_tiling_on_sc=False),
    )
    def gather(idx_hbm, table_hbm, out_hbm, idx_v, rows_v, gsem, osem):
        wid = lax.axis_index("s") * info.num_cores + lax.axis_index("c")
        pltpu.sync_copy(idx_hbm.at[wid], idx_v)
        out_base = wid * rows_per_w

        def start_gather(k):
            b = lax.rem(k, NBUF)
            pltpu.async_copy(table_hbm.at[idx_v.at[k]], rows_v.at[b], gsem.at[b])

        def wait_gather(k):
            b = lax.rem(k, NBUF)
            pltpu.make_async_copy(
                table_hbm.at[idx_v.at[k]], rows_v.at[b], gsem.at[b]
            ).wait()

        def out_ref(k):
            return out_hbm.at[pl.ds(out_base + k * CHUNK, CHUNK)]

        def start_out(k):
            b = lax.rem(k, NBUF)
            pltpu.async_copy(rows_v.at[b], out_ref(k), osem.at[b])

        def wait_out(k):
            b = lax.rem(k, NBUF)
            pltpu.make_async_copy(rows_v.at[b], out_ref(k), osem.at[b]).wait()

        for k in range(NBUF - 1):
            start_gather(k)

        def body(j, _):
            jn = j + NBUF - 1

            @pl.when(jnp.logical_and(jn < chunks_per_w, j > 0))
            def _():
                wait_out(j - 1)

            @pl.when(jn < chunks_per_w)
            def _():
                start_gather(jn)

            wait_gather(j)
            start_out(j)
            return 0

        lax.fori_loop(0, chunks_per_w, body, 0)

        for t in range(NBUF):
            wait_out(chunks_per_w - NBUF + t)

    return gather


def kernel(x, table):
    batch, seq = x.shape
    n_rows = batch * seq
    info = plsc.get_sparse_core_info()
    nw = info.num_cores * info.num_subcores
    x_t = jnp.swapaxes(x, 0, 1)
    idx = x_t.astype(jnp.int32).reshape(nw, n_rows // (nw * CHUNK), CHUNK)
    out = _make_gather(n_rows)(idx, table)
    return jnp.swapaxes(out.reshape(seq, batch, EMBD), 0, 1)
